# Initial kernel scaffold; baseline (speedup 1.0000x reference)
import jax
import jax.numpy as jnp
from jax.experimental import pallas as pl


def kernel(table, y):
    raise NotImplementedError("write your pallas kernel here")


# speedup vs baseline: 1.5011x; 1.5011x over previous
import math

import jax
import jax.numpy as jnp
from jax.experimental import pallas as pl
from jax.experimental.pallas import tpu as pltpu
from jax.experimental.pallas import tpu_sc as plsc

EMB = 128
WINDOW = 128
SCALE = math.sqrt(EMB)


def _gather_scale(table, idx_flat):
    num_indices = idx_flat.shape[1]
    mesh = plsc.VectorSubcoreMesh(core_axis_name="core", subcore_axis_name="subcore")

    @pl.kernel(
        out_type=jax.ShapeDtypeStruct((num_indices, EMB), table.dtype),
        mesh=mesh,
    )
    def kern(x_hbm, i_hbm, o_hbm):
        def body(i_vmem, o_vmem):
            pltpu.sync_copy(x_hbm.at[i_vmem.at[0]], o_vmem)

            @pl.loop(0, WINDOW)
            def _row(r):
                @pl.loop(0, EMB, step=16)
                def _col(c):
                    o_vmem.at[pl.ds(r, 1), pl.ds(c, 16)][...] = (
                        o_vmem.at[pl.ds(r, 1), pl.ds(c, 16)][...] * SCALE
                    )

        pltpu.emit_pipeline(
            body,
            grid=(num_indices // WINDOW,),
            in_specs=[pl.BlockSpec((1, WINDOW), index_map=lambda i: (0, i))],
            out_specs=[pl.BlockSpec((WINDOW, EMB), index_map=lambda i: (i, 0))],
            core_axis_name=("core", "subcore"),
            dimension_semantics=(pltpu.PARALLEL,),
        )(i_hbm, o_hbm)

    return kern(table, idx_flat)


def kernel(table, y):
    b, l = y.shape
    idx = y.reshape(1, b * l).astype(jnp.int32)
    out = _gather_scale(table, idx)
    return out.reshape(b, l, EMB)


# speedup vs baseline: 1.5031x; 1.0013x over previous
import math

import jax
import jax.numpy as jnp
from jax.experimental import pallas as pl
from jax.experimental.pallas import tpu as pltpu
from jax.experimental.pallas import tpu_sc as plsc

EMB = 128
WINDOW = 128
SCALE = math.sqrt(EMB)


def _gather_scale(table, idx_flat):
    num_indices = idx_flat.shape[1]
    mesh = plsc.VectorSubcoreMesh(core_axis_name="core", subcore_axis_name="subcore")

    @pl.kernel(
        out_type=jax.ShapeDtypeStruct((num_indices, EMB), table.dtype),
        mesh=mesh,
    )
    def kern(x_hbm, i_hbm, o_hbm):
        def body(i_vmem, o_vmem):
            pltpu.sync_copy(x_hbm.at[i_vmem.at[0]], o_vmem)

            @pl.loop(0, WINDOW)
            def _row(r):
                for c in range(0, EMB, 16):
                    o_vmem.at[pl.ds(r, 1), pl.ds(c, 16)][...] = (
                        o_vmem.at[pl.ds(r, 1), pl.ds(c, 16)][...] * SCALE
                    )

        pltpu.emit_pipeline(
            body,
            grid=(num_indices // WINDOW,),
            in_specs=[pl.BlockSpec((1, WINDOW), index_map=lambda i: (0, i))],
            out_specs=[pl.BlockSpec((WINDOW, EMB), index_map=lambda i: (i, 0))],
            core_axis_name=("core", "subcore"),
            dimension_semantics=(pltpu.PARALLEL,),
        )(i_hbm, o_hbm)

    return kern(table, idx_flat)


def kernel(table, y):
    b, l = y.shape
    idx = y.reshape(1, b * l).astype(jnp.int32)
    out = _gather_scale(table, idx)
    return out.reshape(b, l, EMB)


# speedup vs baseline: 2.3626x; 1.5718x over previous
import math

import jax
import jax.numpy as jnp
from jax.experimental import pallas as pl
from jax.experimental.pallas import tpu as pltpu
from jax.experimental.pallas import tpu_sc as plsc

EMB = 128
WINDOW = 128
SCALE = math.sqrt(EMB)
ROWS_PER_BLOCK = 2000


def _scale_table(table):
    def body(x_ref, o_ref):
        o_ref[...] = x_ref[...] * SCALE

    return pl.pallas_call(
        body,
        out_shape=jax.ShapeDtypeStruct(table.shape, table.dtype),
        grid=(table.shape[0] // ROWS_PER_BLOCK,),
        in_specs=[pl.BlockSpec((ROWS_PER_BLOCK, EMB), lambda i: (i, 0))],
        out_specs=pl.BlockSpec((ROWS_PER_BLOCK, EMB), lambda i: (i, 0)),
    )(table)


def _gather(table, idx_flat):
    num_indices = idx_flat.shape[1]
    mesh = plsc.VectorSubcoreMesh(core_axis_name="core", subcore_axis_name="subcore")

    @pl.kernel(
        out_type=jax.ShapeDtypeStruct((num_indices, EMB), table.dtype),
        mesh=mesh,
    )
    def kern(x_hbm, i_hbm, o_hbm):
        def body(i_vmem, o_vmem):
            pltpu.sync_copy(x_hbm.at[i_vmem.at[0]], o_vmem)

        pltpu.emit_pipeline(
            body,
            grid=(num_indices // WINDOW,),
            in_specs=[pl.BlockSpec((1, WINDOW), index_map=lambda i: (0, i))],
            out_specs=[pl.BlockSpec((WINDOW, EMB), index_map=lambda i: (i, 0))],
            core_axis_name=("core", "subcore"),
            dimension_semantics=(pltpu.PARALLEL,),
        )(i_hbm, o_hbm)

    return kern(table, idx_flat)


def kernel(table, y):
    b, l = y.shape
    idx = y.reshape(1, b * l).astype(jnp.int32)
    out = _gather(_scale_table(table), idx)
    return out.reshape(b, l, EMB)


# speedup vs baseline: 2.4816x; 1.0504x over previous
import math

import jax
import jax.numpy as jnp
from jax.experimental import pallas as pl
from jax.experimental.pallas import tpu as pltpu
from jax.experimental.pallas import tpu_sc as plsc

EMB = 128
WINDOW = 256
SCALE = math.sqrt(EMB)
ROWS_PER_BLOCK = 2000


def _scale_table(table):
    def body(x_ref, o_ref):
        o_ref[...] = x_ref[...] * SCALE

    return pl.pallas_call(
        body,
        out_shape=jax.ShapeDtypeStruct(table.shape, table.dtype),
        grid=(table.shape[0] // ROWS_PER_BLOCK,),
        in_specs=[pl.BlockSpec((ROWS_PER_BLOCK, EMB), lambda i: (i, 0))],
        out_specs=pl.BlockSpec((ROWS_PER_BLOCK, EMB), lambda i: (i, 0)),
    )(table)


def _gather(table, idx_flat):
    num_indices = idx_flat.shape[1]
    mesh = plsc.VectorSubcoreMesh(core_axis_name="core", subcore_axis_name="subcore")

    @pl.kernel(
        out_type=jax.ShapeDtypeStruct((num_indices, EMB), table.dtype),
        mesh=mesh,
    )
    def kern(x_hbm, i_hbm, o_hbm):
        def body(i_vmem, o_vmem):
            pltpu.sync_copy(x_hbm.at[i_vmem.at[0]], o_vmem)

        pltpu.emit_pipeline(
            body,
            grid=(num_indices // WINDOW,),
            in_specs=[pl.BlockSpec((1, WINDOW), index_map=lambda i: (0, i))],
            out_specs=[pl.BlockSpec((WINDOW, EMB), index_map=lambda i: (i, 0))],
            core_axis_name=("core", "subcore"),
            dimension_semantics=(pltpu.PARALLEL,),
        )(i_hbm, o_hbm)

    return kern(table, idx_flat)


def kernel(table, y):
    b, l = y.shape
    idx = y.reshape(1, b * l).astype(jnp.int32)
    out = _gather(_scale_table(table), idx)
    return out.reshape(b, l, EMB)


# speedup vs baseline: 3.0197x; 1.2168x over previous
import math

import jax
import jax.numpy as jnp
from jax.experimental import pallas as pl
from jax.experimental.pallas import tpu as pltpu
from jax.experimental.pallas import tpu_sc as plsc

EMB = 128
WINDOW = 256
SCALE = math.sqrt(EMB)
ROWS_PER_BLOCK = 2000


def _scale_table(table):
    def body(x_ref, o_ref):
        o_ref[...] = x_ref[...] * SCALE

    return pl.pallas_call(
        body,
        out_shape=jax.ShapeDtypeStruct(table.shape, table.dtype),
        grid=(table.shape[0] // ROWS_PER_BLOCK,),
        in_specs=[pl.BlockSpec((ROWS_PER_BLOCK, EMB), lambda i: (i, 0))],
        out_specs=pl.BlockSpec((ROWS_PER_BLOCK, EMB), lambda i: (i, 0)),
    )(table)


def _gather(table, idx3):
    b = idx3.shape[0]
    l = idx3.shape[2]
    mesh = plsc.VectorSubcoreMesh(core_axis_name="core", subcore_axis_name="subcore")

    @pl.kernel(
        out_type=jax.ShapeDtypeStruct((b, l, EMB), table.dtype),
        mesh=mesh,
    )
    def kern(x_hbm, i_hbm, o_hbm):
        def body(i_vmem, o_vmem):
            pltpu.sync_copy(x_hbm.at[i_vmem.at[0, 0]], o_vmem.at[0])

        pltpu.emit_pipeline(
            body,
            grid=(b,),
            in_specs=[pl.BlockSpec((1, 1, l), index_map=lambda i: (i, 0, 0))],
            out_specs=[pl.BlockSpec((1, l, EMB), index_map=lambda i: (i, 0, 0))],
            core_axis_name=("core", "subcore"),
            dimension_semantics=(pltpu.PARALLEL,),
        )(i_hbm, o_hbm)

    return kern(table, idx3)


def kernel(table, y):
    b, l = y.shape
    idx = y.reshape(b, 1, l).astype(jnp.int32)
    return _gather(_scale_table(table), idx)


# speedup vs baseline: 3.0602x; 1.0134x over previous
import math

import jax
import jax.numpy as jnp
from jax.experimental import pallas as pl
from jax.experimental.pallas import tpu as pltpu
from jax.experimental.pallas import tpu_sc as plsc

EMB = 128
WINDOW = 256
SCALE = math.sqrt(EMB)
ROWS_PER_BLOCK = 2000


def _scale_table(table):
    def body(x_ref, o_ref):
        o_ref[...] = x_ref[...] * SCALE

    return pl.pallas_call(
        body,
        out_shape=jax.ShapeDtypeStruct(table.shape, table.dtype),
        grid=(table.shape[0] // ROWS_PER_BLOCK,),
        in_specs=[pl.BlockSpec((ROWS_PER_BLOCK, EMB), lambda i: (i, 0))],
        out_specs=pl.BlockSpec((ROWS_PER_BLOCK, EMB), lambda i: (i, 0)),
    )(table)


def _gather(table, idx3):
    b = idx3.shape[0]
    l = idx3.shape[2]
    mesh = plsc.VectorSubcoreMesh(core_axis_name="core", subcore_axis_name="subcore")

    bb = 8

    @pl.kernel(
        out_type=jax.ShapeDtypeStruct((b, l, EMB), table.dtype),
        mesh=mesh,
    )
    def kern(x_hbm, i_hbm, o_hbm):
        def body(i_vmem, o_vmem):
            for j in range(bb):
                pltpu.sync_copy(x_hbm.at[i_vmem.at[j, 0]], o_vmem.at[j])

        pltpu.emit_pipeline(
            body,
            grid=(b // bb,),
            in_specs=[pl.BlockSpec((bb, 1, l), index_map=lambda i: (i, 0, 0))],
            out_specs=[pl.BlockSpec((bb, l, EMB), index_map=lambda i: (i, 0, 0))],
            core_axis_name=("core", "subcore"),
            dimension_semantics=(pltpu.PARALLEL,),
        )(i_hbm, o_hbm)

    return kern(table, idx3)


def kernel(table, y):
    b, l = y.shape
    idx = y.reshape(b, 1, l).astype(jnp.int32)
    return _gather(_scale_table(table), idx)
